# baseline (speedup 1.0000x reference)
import functools

import jax
import jax.numpy as jnp
from jax import lax
from jax.experimental import pallas as pl
from jax.experimental.pallas import tpu as pltpu
from jax.experimental.pallas import tpu_sc as plsc

NC = 2
NS = 16
NW = NC * NS
L = 16

NROWS = 1000000
BATCH = 16384
D = 64
W2 = 2 * D
B_PER_W = BATCH // NW
CHUNK = 128
NCHUNK = B_PER_W // CHUNK
NG = CHUNK // L


def _fcf_body(user_hbm, item_hbm, utab_hbm, itab_hbm, params_hbm, out_hbm,
              uidx_v, iidx_v, ublk_v, iblk_v, ubuf_v, ibuf_v, params_v,
              out_v, mat_v, sems):
    wid = lax.axis_index("s") * NC + lax.axis_index("c")
    base = wid * B_PER_W

    pltpu.sync_copy(user_hbm.at[pl.ds(base, B_PER_W)], uidx_v)
    pltpu.sync_copy(item_hbm.at[pl.ds(base, B_PER_W)], iidx_v)
    pltpu.sync_copy(params_hbm, params_v)

    for g in range(B_PER_W // L):
        sl = pl.ds(g * L, L)
        ublk_v[sl] = lax.shift_right_logical(uidx_v[sl], 1)
        iblk_v[sl] = lax.shift_right_logical(iidx_v[sl], 1)

    def fire(j, slot):
        pltpu.async_copy(
            utab_hbm.at[ublk_v.at[pl.ds(j * CHUNK, CHUNK)]],
            ubuf_v.at[slot], sems.at[slot, 0])
        pltpu.async_copy(
            itab_hbm.at[iblk_v.at[pl.ds(j * CHUNK, CHUNK)]],
            ibuf_v.at[slot], sems.at[slot, 1])

    def drain(slot):
        pltpu.make_async_copy(
            utab_hbm.at[ublk_v.at[pl.ds(0, CHUNK)]],
            ubuf_v.at[slot], sems.at[slot, 0]).wait()
        pltpu.make_async_copy(
            itab_hbm.at[iblk_v.at[pl.ds(0, CHUNK)]],
            ibuf_v.at[slot], sems.at[slot, 1]).wait()

    w0 = params_v[pl.ds(0, L)]
    w1 = params_v[pl.ds(L, L)]
    w2 = params_v[pl.ds(2 * L, L)]
    w3 = params_v[pl.ds(3 * L, L)]
    bias_splat = jnp.full((L,), params_v[pl.ds(D, L)][0], jnp.float32)
    iota = lax.iota(jnp.int32, L)
    one = jnp.full((L,), 1, jnp.int32)

    def compute(j, slot):
        def grp(g, carry):
            sl = pl.ds(j * CHUNK + g * L, L)
            uoffv = lax.bitwise_and(uidx_v[sl], one) * D
            ioffv = lax.bitwise_and(iidx_v[sl], one) * D
            for bb in range(L):
                b = g * L + bb
                uo = uoffv[bb]
                io = ioffv[bb]
                acc = (ubuf_v[slot, b, pl.ds(uo, L)]
                       * ibuf_v[slot, b, pl.ds(io, L)] * w0)
                acc = acc + (ubuf_v[slot, b, pl.ds(uo + L, L)]
                             * ibuf_v[slot, b, pl.ds(io + L, L)] * w1)
                acc = acc + (ubuf_v[slot, b, pl.ds(uo + 2 * L, L)]
                             * ibuf_v[slot, b, pl.ds(io + 2 * L, L)] * w2)
                acc = acc + (ubuf_v[slot, b, pl.ds(uo + 3 * L, L)]
                             * ibuf_v[slot, b, pl.ds(io + 3 * L, L)] * w3)
                mat_v[bb, :] = acc
            colsum = bias_splat
            for c in range(L):
                colsum = colsum + plsc.load_gather(
                    mat_v, [iota, jnp.full((L,), c, jnp.int32)])
            out_v[pl.ds(j * CHUNK + g * L, L)] = colsum
            return carry

        lax.fori_loop(0, NG, grp, 0)

    fire(0, 0)
    for j in range(NCHUNK):
        slot = j % 2
        if j + 1 < NCHUNK:
            fire(j + 1, 1 - slot)
        drain(slot)
        compute(j, slot)

    pltpu.sync_copy(out_v, out_hbm.at[pl.ds(base, B_PER_W)])


def kernel(user, item, users_embeddings, items_embeddings, affine_w, affine_b):
    user_i = user.astype(jnp.int32)
    item_i = item.astype(jnp.int32)
    utab2 = users_embeddings.reshape(NROWS // 2, W2)
    itab2 = items_embeddings.reshape(NROWS // 2, W2)
    params = jnp.concatenate(
        [affine_w.reshape(-1), affine_b.reshape(-1),
         jnp.zeros((15,), jnp.float32)])

    mesh = plsc.VectorSubcoreMesh(core_axis_name="c", subcore_axis_name="s")
    fcf = functools.partial(
        pl.kernel,
        mesh=mesh,
        compiler_params=pltpu.CompilerParams(
            needs_layout_passes=False, use_tc_tiling_on_sc=True),
        out_type=jax.ShapeDtypeStruct((BATCH,), jnp.float32),
        scratch_types=[
            pltpu.VMEM((B_PER_W,), jnp.int32),
            pltpu.VMEM((B_PER_W,), jnp.int32),
            pltpu.VMEM((B_PER_W,), jnp.int32),
            pltpu.VMEM((B_PER_W,), jnp.int32),
            pltpu.VMEM((2, CHUNK, W2), jnp.float32),
            pltpu.VMEM((2, CHUNK, W2), jnp.float32),
            pltpu.VMEM((80,), jnp.float32),
            pltpu.VMEM((B_PER_W,), jnp.float32),
            pltpu.VMEM((L, L), jnp.float32),
            pltpu.SemaphoreType.DMA((2, 2)),
        ],
    )(_fcf_body)
    return fcf(user_i, item_i, utab2, itab2, params)


# speedup vs baseline: 1.1640x; 1.1640x over previous
import functools

import jax
import jax.numpy as jnp
from jax import lax
from jax.experimental import pallas as pl
from jax.experimental.pallas import tpu as pltpu
from jax.experimental.pallas import tpu_sc as plsc

NC = 2
NS = 16
NW = NC * NS
L = 16

NROWS = 1000000
BATCH = 16384
D = 64
W2 = 2 * D
B_PER_W = BATCH // NW
CHUNK = 128
NCHUNK = B_PER_W // CHUNK
NG = CHUNK // L

BLK_R = 2048
TGRID = (NROWS + BLK_R - 1) // BLK_R


def _transpose_body(in_ref, o_ref):
    o_ref[:, :D] = in_ref[...].T


def _relayout(tabT):
    return pl.pallas_call(
        _transpose_body,
        grid=(TGRID,),
        in_specs=[pl.BlockSpec((D, BLK_R), lambda j: (0, j))],
        out_specs=pl.BlockSpec((BLK_R, W2), lambda j: (j, 0)),
        out_shape=jax.ShapeDtypeStruct((NROWS, W2), jnp.float32),
    )(tabT)


def _fcf_body(user_hbm, item_hbm, utab_hbm, itab_hbm, params_hbm, out_hbm,
              uidx_v, iidx_v, ubuf_v, ibuf_v, params_v, out_v, mat_v, sems):
    wid = lax.axis_index("s") * NC + lax.axis_index("c")
    base = wid * B_PER_W

    pltpu.sync_copy(user_hbm.at[pl.ds(base, B_PER_W)], uidx_v)
    pltpu.sync_copy(item_hbm.at[pl.ds(base, B_PER_W)], iidx_v)
    pltpu.sync_copy(params_hbm, params_v)

    def fire(j, slot):
        pltpu.async_copy(
            utab_hbm.at[uidx_v.at[pl.ds(j * CHUNK, CHUNK)]],
            ubuf_v.at[slot], sems.at[slot, 0])
        pltpu.async_copy(
            itab_hbm.at[iidx_v.at[pl.ds(j * CHUNK, CHUNK)]],
            ibuf_v.at[slot], sems.at[slot, 1])

    def drain(slot):
        pltpu.make_async_copy(
            utab_hbm.at[uidx_v.at[pl.ds(0, CHUNK)]],
            ubuf_v.at[slot], sems.at[slot, 0]).wait()
        pltpu.make_async_copy(
            itab_hbm.at[iidx_v.at[pl.ds(0, CHUNK)]],
            ibuf_v.at[slot], sems.at[slot, 1]).wait()

    w0 = params_v[pl.ds(0, L)]
    w1 = params_v[pl.ds(L, L)]
    w2 = params_v[pl.ds(2 * L, L)]
    w3 = params_v[pl.ds(3 * L, L)]
    bias_splat = jnp.full((L,), params_v[pl.ds(D, L)][0], jnp.float32)
    iota = lax.iota(jnp.int32, L)

    def compute(j, slot):
        def grp(g, carry):
            for bb in range(L):
                b = g * L + bb
                acc = (ubuf_v[slot, b, pl.ds(0, L)]
                       * ibuf_v[slot, b, pl.ds(0, L)] * w0)
                acc = acc + (ubuf_v[slot, b, pl.ds(L, L)]
                             * ibuf_v[slot, b, pl.ds(L, L)] * w1)
                acc = acc + (ubuf_v[slot, b, pl.ds(2 * L, L)]
                             * ibuf_v[slot, b, pl.ds(2 * L, L)] * w2)
                acc = acc + (ubuf_v[slot, b, pl.ds(3 * L, L)]
                             * ibuf_v[slot, b, pl.ds(3 * L, L)] * w3)
                mat_v[bb, :] = acc
            colsum = bias_splat
            for c in range(L):
                colsum = colsum + plsc.load_gather(
                    mat_v, [iota, jnp.full((L,), c, jnp.int32)])
            out_v[pl.ds(j * CHUNK + g * L, L)] = colsum
            return carry

        lax.fori_loop(0, NG, grp, 0)

    fire(0, 0)
    for j in range(NCHUNK):
        slot = j % 2
        if j + 1 < NCHUNK:
            fire(j + 1, 1 - slot)
        drain(slot)
        compute(j, slot)

    pltpu.sync_copy(out_v, out_hbm.at[pl.ds(base, B_PER_W)])


def kernel(user, item, users_embeddings, items_embeddings, affine_w, affine_b):
    user_i = user.astype(jnp.int32)
    item_i = item.astype(jnp.int32)
    utab2 = _relayout(users_embeddings.T)
    itab2 = _relayout(items_embeddings.T)
    params = jnp.concatenate(
        [affine_w.reshape(-1), affine_b.reshape(-1),
         jnp.zeros((15,), jnp.float32)])

    mesh = plsc.VectorSubcoreMesh(core_axis_name="c", subcore_axis_name="s")
    fcf = functools.partial(
        pl.kernel,
        mesh=mesh,
        compiler_params=pltpu.CompilerParams(
            needs_layout_passes=False, use_tc_tiling_on_sc=True),
        out_type=jax.ShapeDtypeStruct((BATCH,), jnp.float32),
        scratch_types=[
            pltpu.VMEM((B_PER_W,), jnp.int32),
            pltpu.VMEM((B_PER_W,), jnp.int32),
            pltpu.VMEM((2, CHUNK, W2), jnp.float32),
            pltpu.VMEM((2, CHUNK, W2), jnp.float32),
            pltpu.VMEM((80,), jnp.float32),
            pltpu.VMEM((B_PER_W,), jnp.float32),
            pltpu.VMEM((L, L), jnp.float32),
            pltpu.SemaphoreType.DMA((2, 2)),
        ],
    )(_fcf_body)
    return fcf(user_i, item_i, utab2, itab2, params)


# speedup vs baseline: 2.3607x; 2.0281x over previous
import functools

import jax
import jax.numpy as jnp
from jax import lax
from jax.experimental import pallas as pl
from jax.experimental.pallas import tpu as pltpu
from jax.experimental.pallas import tpu_sc as plsc

NC = 2
NS = 16
NW = NC * NS
L = 16

NROWS = 1000000
BATCH = 16384
D = 64
W2 = 2 * D
B_PER_W = BATCH // NW
CHUNK = 128
NCHUNK = B_PER_W // CHUNK
NG = CHUNK // L

BLK_R = 8192
TGRID = (NROWS + BLK_R - 1) // BLK_R


def _transpose_body(u_ref, i_ref, o_ref):
    o_ref[:, :D] = u_ref[...].T
    o_ref[:, D:] = i_ref[...].T


def _relayout(utabT, itabT):
    return pl.pallas_call(
        _transpose_body,
        grid=(TGRID,),
        in_specs=[pl.BlockSpec((D, BLK_R), lambda j: (0, j)),
                  pl.BlockSpec((D, BLK_R), lambda j: (0, j))],
        out_specs=pl.BlockSpec((BLK_R, W2), lambda j: (j, 0)),
        out_shape=jax.ShapeDtypeStruct((NROWS, W2), jnp.float32),
    )(utabT, itabT)


def _fcf_body(user_hbm, item_hbm, tab_hbm, params_hbm, out_hbm,
              uidx_v, iidx_v, ubuf_v, ibuf_v, params_v, out_v, mat_v, sems):
    wid = lax.axis_index("s") * NC + lax.axis_index("c")
    base = wid * B_PER_W

    pltpu.sync_copy(user_hbm.at[pl.ds(base, B_PER_W)], uidx_v)
    pltpu.sync_copy(item_hbm.at[pl.ds(base, B_PER_W)], iidx_v)
    pltpu.sync_copy(params_hbm, params_v)

    def fire(j, slot):
        pltpu.async_copy(
            tab_hbm.at[uidx_v.at[pl.ds(j * CHUNK, CHUNK)]],
            ubuf_v.at[slot], sems.at[slot, 0])
        pltpu.async_copy(
            tab_hbm.at[iidx_v.at[pl.ds(j * CHUNK, CHUNK)]],
            ibuf_v.at[slot], sems.at[slot, 1])

    def drain(slot):
        pltpu.make_async_copy(
            tab_hbm.at[uidx_v.at[pl.ds(0, CHUNK)]],
            ubuf_v.at[slot], sems.at[slot, 0]).wait()
        pltpu.make_async_copy(
            tab_hbm.at[iidx_v.at[pl.ds(0, CHUNK)]],
            ibuf_v.at[slot], sems.at[slot, 1]).wait()

    w0 = params_v[pl.ds(0, L)]
    w1 = params_v[pl.ds(L, L)]
    w2 = params_v[pl.ds(2 * L, L)]
    w3 = params_v[pl.ds(3 * L, L)]
    bias_splat = jnp.full((L,), params_v[pl.ds(D, L)][0], jnp.float32)
    iota = lax.iota(jnp.int32, L)

    def compute(j, slot):
        def grp(g, carry):
            for bb in range(L):
                b = g * L + bb
                acc = (ubuf_v[slot, b, pl.ds(0, L)]
                       * ibuf_v[slot, b, pl.ds(D, L)] * w0)
                acc = acc + (ubuf_v[slot, b, pl.ds(L, L)]
                             * ibuf_v[slot, b, pl.ds(D + L, L)] * w1)
                acc = acc + (ubuf_v[slot, b, pl.ds(2 * L, L)]
                             * ibuf_v[slot, b, pl.ds(D + 2 * L, L)] * w2)
                acc = acc + (ubuf_v[slot, b, pl.ds(3 * L, L)]
                             * ibuf_v[slot, b, pl.ds(D + 3 * L, L)] * w3)
                mat_v[bb, :] = acc
            colsum = bias_splat
            for c in range(L):
                colsum = colsum + plsc.load_gather(
                    mat_v, [iota, jnp.full((L,), c, jnp.int32)])
            out_v[pl.ds(j * CHUNK + g * L, L)] = colsum
            return carry

        lax.fori_loop(0, NG, grp, 0)

    fire(0, 0)
    for j in range(NCHUNK):
        slot = j % 2
        if j + 1 < NCHUNK:
            fire(j + 1, 1 - slot)
        drain(slot)
        compute(j, slot)

    pltpu.sync_copy(out_v, out_hbm.at[pl.ds(base, B_PER_W)])


def kernel(user, item, users_embeddings, items_embeddings, affine_w, affine_b):
    user_i = user.astype(jnp.int32)
    item_i = item.astype(jnp.int32)
    tab = _relayout(users_embeddings.T, items_embeddings.T)
    params = jnp.concatenate(
        [affine_w.reshape(-1), affine_b.reshape(-1),
         jnp.zeros((15,), jnp.float32)])

    mesh = plsc.VectorSubcoreMesh(core_axis_name="c", subcore_axis_name="s")
    fcf = functools.partial(
        pl.kernel,
        mesh=mesh,
        compiler_params=pltpu.CompilerParams(
            needs_layout_passes=False, use_tc_tiling_on_sc=True),
        out_type=jax.ShapeDtypeStruct((BATCH,), jnp.float32),
        scratch_types=[
            pltpu.VMEM((B_PER_W,), jnp.int32),
            pltpu.VMEM((B_PER_W,), jnp.int32),
            pltpu.VMEM((2, CHUNK, W2), jnp.float32),
            pltpu.VMEM((2, CHUNK, W2), jnp.float32),
            pltpu.VMEM((80,), jnp.float32),
            pltpu.VMEM((B_PER_W,), jnp.float32),
            pltpu.VMEM((L, L), jnp.float32),
            pltpu.SemaphoreType.DMA((2, 2)),
        ],
    )(_fcf_body)
    return fcf(user_i, item_i, tab, params)


# speedup vs baseline: 2.5157x; 1.0657x over previous
import functools

import jax
import jax.numpy as jnp
from jax import lax
from jax.experimental import pallas as pl
from jax.experimental.pallas import tpu as pltpu
from jax.experimental.pallas import tpu_sc as plsc

NC = 2
NS = 16
NW = NC * NS
L = 16

NROWS = 1000000
BATCH = 16384
D = 64
W2 = 2 * D
B_PER_W = BATCH // NW
CHUNK = 128
NCHUNK = B_PER_W // CHUNK
NG = CHUNK // L

BLK_R = 16384
TGRID = (NROWS + BLK_R - 1) // BLK_R


def _transpose_body(u_ref, i_ref, o_ref):
    o_ref[:, :D] = u_ref[...].T
    o_ref[:, D:] = i_ref[...].T


def _relayout(utabT, itabT):
    return pl.pallas_call(
        _transpose_body,
        grid=(TGRID,),
        in_specs=[pl.BlockSpec((D, BLK_R), lambda j: (0, j)),
                  pl.BlockSpec((D, BLK_R), lambda j: (0, j))],
        out_specs=pl.BlockSpec((BLK_R, W2), lambda j: (j, 0)),
        out_shape=jax.ShapeDtypeStruct((NROWS, W2), jnp.float32),
    )(utabT, itabT)


def _fcf_body(user_hbm, item_hbm, tab_hbm, params_hbm, out_hbm,
              uidx_v, iidx_v, ubuf_v, ibuf_v, params_v, out_v, mat_v, sems):
    wid = lax.axis_index("s") * NC + lax.axis_index("c")
    base = wid * B_PER_W

    pltpu.sync_copy(user_hbm.at[pl.ds(base, B_PER_W)], uidx_v)
    pltpu.sync_copy(item_hbm.at[pl.ds(base, B_PER_W)], iidx_v)
    pltpu.sync_copy(params_hbm, params_v)

    def fire(j, slot):
        pltpu.async_copy(
            tab_hbm.at[uidx_v.at[pl.ds(j * CHUNK, CHUNK)]],
            ubuf_v.at[slot], sems.at[slot, 0])
        pltpu.async_copy(
            tab_hbm.at[iidx_v.at[pl.ds(j * CHUNK, CHUNK)]],
            ibuf_v.at[slot], sems.at[slot, 1])

    def drain(slot):
        pltpu.make_async_copy(
            tab_hbm.at[uidx_v.at[pl.ds(0, CHUNK)]],
            ubuf_v.at[slot], sems.at[slot, 0]).wait()
        pltpu.make_async_copy(
            tab_hbm.at[iidx_v.at[pl.ds(0, CHUNK)]],
            ibuf_v.at[slot], sems.at[slot, 1]).wait()

    w0 = params_v[pl.ds(0, L)]
    w1 = params_v[pl.ds(L, L)]
    w2 = params_v[pl.ds(2 * L, L)]
    w3 = params_v[pl.ds(3 * L, L)]
    bias_splat = jnp.full((L,), params_v[pl.ds(D, L)][0], jnp.float32)
    iota = lax.iota(jnp.int32, L)

    def compute(j, slot):
        def grp(g, carry):
            for bb in range(L):
                b = g * L + bb
                acc = (ubuf_v[slot, b, pl.ds(0, L)]
                       * ibuf_v[slot, b, pl.ds(D, L)] * w0)
                acc = acc + (ubuf_v[slot, b, pl.ds(L, L)]
                             * ibuf_v[slot, b, pl.ds(D + L, L)] * w1)
                acc = acc + (ubuf_v[slot, b, pl.ds(2 * L, L)]
                             * ibuf_v[slot, b, pl.ds(D + 2 * L, L)] * w2)
                acc = acc + (ubuf_v[slot, b, pl.ds(3 * L, L)]
                             * ibuf_v[slot, b, pl.ds(D + 3 * L, L)] * w3)
                mat_v[bb, :] = acc
            colsum = bias_splat
            for c in range(L):
                colsum = colsum + plsc.load_gather(
                    mat_v, [iota, jnp.full((L,), c, jnp.int32)])
            out_v[pl.ds(j * CHUNK + g * L, L)] = colsum
            return carry

        lax.fori_loop(0, NG, grp, 0)

    fire(0, 0)
    for j in range(NCHUNK):
        slot = j % 2
        if j + 1 < NCHUNK:
            fire(j + 1, 1 - slot)
        drain(slot)
        compute(j, slot)

    pltpu.sync_copy(out_v, out_hbm.at[pl.ds(base, B_PER_W)])


def kernel(user, item, users_embeddings, items_embeddings, affine_w, affine_b):
    user_i = user.astype(jnp.int32)
    item_i = item.astype(jnp.int32)
    tab = _relayout(users_embeddings.T, items_embeddings.T)
    params = jnp.concatenate(
        [affine_w.reshape(-1), affine_b.reshape(-1),
         jnp.zeros((15,), jnp.float32)])

    mesh = plsc.VectorSubcoreMesh(core_axis_name="c", subcore_axis_name="s")
    fcf = functools.partial(
        pl.kernel,
        mesh=mesh,
        compiler_params=pltpu.CompilerParams(
            needs_layout_passes=False, use_tc_tiling_on_sc=True),
        out_type=jax.ShapeDtypeStruct((BATCH,), jnp.float32),
        scratch_types=[
            pltpu.VMEM((B_PER_W,), jnp.int32),
            pltpu.VMEM((B_PER_W,), jnp.int32),
            pltpu.VMEM((2, CHUNK, W2), jnp.float32),
            pltpu.VMEM((2, CHUNK, W2), jnp.float32),
            pltpu.VMEM((80,), jnp.float32),
            pltpu.VMEM((B_PER_W,), jnp.float32),
            pltpu.VMEM((L, L), jnp.float32),
            pltpu.SemaphoreType.DMA((2, 2)),
        ],
    )(_fcf_body)
    return fcf(user_i, item_i, tab, params)


# speedup vs baseline: 2.5262x; 1.0042x over previous
import functools

import jax
import jax.numpy as jnp
from jax import lax
from jax.experimental import pallas as pl
from jax.experimental.pallas import tpu as pltpu
from jax.experimental.pallas import tpu_sc as plsc

NC = 2
NS = 16
NW = NC * NS
L = 16

NROWS = 1000000
BATCH = 16384
D = 64
W2 = 2 * D
B_PER_W = BATCH // NW
CHUNK = 128
NCHUNK = B_PER_W // CHUNK
NG = CHUNK // L

BLK_R = 23552
TGRID = (NROWS + BLK_R - 1) // BLK_R


def _transpose_body(u_ref, i_ref, o_ref):
    o_ref[:, :D] = u_ref[...].T
    o_ref[:, D:] = i_ref[...].T


def _relayout(utabT, itabT):
    return pl.pallas_call(
        _transpose_body,
        grid=(TGRID,),
        in_specs=[pl.BlockSpec((D, BLK_R), lambda j: (0, j)),
                  pl.BlockSpec((D, BLK_R), lambda j: (0, j))],
        out_specs=pl.BlockSpec((BLK_R, W2), lambda j: (j, 0)),
        out_shape=jax.ShapeDtypeStruct((NROWS, W2), jnp.float32),
    )(utabT, itabT)


def _fcf_body(user_hbm, item_hbm, tab_hbm, params_hbm, out_hbm,
              uidx_v, iidx_v, ubuf_v, ibuf_v, params_v, out_v, mat_v, sems):
    wid = lax.axis_index("s") * NC + lax.axis_index("c")
    base = wid * B_PER_W

    pltpu.sync_copy(user_hbm.at[pl.ds(base, B_PER_W)], uidx_v)
    pltpu.sync_copy(item_hbm.at[pl.ds(base, B_PER_W)], iidx_v)
    pltpu.sync_copy(params_hbm, params_v)

    def fire(j, slot):
        pltpu.async_copy(
            tab_hbm.at[uidx_v.at[pl.ds(j * CHUNK, CHUNK)]],
            ubuf_v.at[slot], sems.at[slot, 0])
        pltpu.async_copy(
            tab_hbm.at[iidx_v.at[pl.ds(j * CHUNK, CHUNK)]],
            ibuf_v.at[slot], sems.at[slot, 1])

    def drain(slot):
        pltpu.make_async_copy(
            tab_hbm.at[uidx_v.at[pl.ds(0, CHUNK)]],
            ubuf_v.at[slot], sems.at[slot, 0]).wait()
        pltpu.make_async_copy(
            tab_hbm.at[iidx_v.at[pl.ds(0, CHUNK)]],
            ibuf_v.at[slot], sems.at[slot, 1]).wait()

    w0 = params_v[pl.ds(0, L)]
    w1 = params_v[pl.ds(L, L)]
    w2 = params_v[pl.ds(2 * L, L)]
    w3 = params_v[pl.ds(3 * L, L)]
    bias_splat = jnp.full((L,), params_v[pl.ds(D, L)][0], jnp.float32)
    iota = lax.iota(jnp.int32, L)

    def compute(j, slot):
        def grp(g, carry):
            for bb in range(L):
                b = g * L + bb
                acc = (ubuf_v[slot, b, pl.ds(0, L)]
                       * ibuf_v[slot, b, pl.ds(D, L)] * w0)
                acc = acc + (ubuf_v[slot, b, pl.ds(L, L)]
                             * ibuf_v[slot, b, pl.ds(D + L, L)] * w1)
                acc = acc + (ubuf_v[slot, b, pl.ds(2 * L, L)]
                             * ibuf_v[slot, b, pl.ds(D + 2 * L, L)] * w2)
                acc = acc + (ubuf_v[slot, b, pl.ds(3 * L, L)]
                             * ibuf_v[slot, b, pl.ds(D + 3 * L, L)] * w3)
                mat_v[bb, :] = acc
            colsum = bias_splat
            for c in range(L):
                colsum = colsum + plsc.load_gather(
                    mat_v, [iota, jnp.full((L,), c, jnp.int32)])
            out_v[pl.ds(j * CHUNK + g * L, L)] = colsum
            return carry

        lax.fori_loop(0, NG, grp, 0)

    fire(0, 0)
    for j in range(NCHUNK):
        slot = j % 2
        if j + 1 < NCHUNK:
            fire(j + 1, 1 - slot)
        drain(slot)
        compute(j, slot)

    pltpu.sync_copy(out_v, out_hbm.at[pl.ds(base, B_PER_W)])


def kernel(user, item, users_embeddings, items_embeddings, affine_w, affine_b):
    user_i = user.astype(jnp.int32)
    item_i = item.astype(jnp.int32)
    tab = _relayout(users_embeddings.T, items_embeddings.T)
    params = jnp.concatenate(
        [affine_w.reshape(-1), affine_b.reshape(-1),
         jnp.zeros((15,), jnp.float32)])

    mesh = plsc.VectorSubcoreMesh(core_axis_name="c", subcore_axis_name="s")
    fcf = functools.partial(
        pl.kernel,
        mesh=mesh,
        compiler_params=pltpu.CompilerParams(
            needs_layout_passes=False, use_tc_tiling_on_sc=True),
        out_type=jax.ShapeDtypeStruct((BATCH,), jnp.float32),
        scratch_types=[
            pltpu.VMEM((B_PER_W,), jnp.int32),
            pltpu.VMEM((B_PER_W,), jnp.int32),
            pltpu.VMEM((2, CHUNK, W2), jnp.float32),
            pltpu.VMEM((2, CHUNK, W2), jnp.float32),
            pltpu.VMEM((80,), jnp.float32),
            pltpu.VMEM((B_PER_W,), jnp.float32),
            pltpu.VMEM((L, L), jnp.float32),
            pltpu.SemaphoreType.DMA((2, 2)),
        ],
    )(_fcf_body)
    return fcf(user_i, item_i, tab, params)
